# baseline (speedup 1.0000x reference)
import functools

import jax
import jax.numpy as jnp
from jax import lax
from jax.experimental import pallas as pl
from jax.experimental.pallas import tpu as pltpu, tpu_sc as plsc

N = 10000
C = 256
E = 160000
CH = 128
NC, NS, K = 2, 16, 128
NW = NC * NS

NB_A = 40
EPA = NW * NB_A * K
G16 = 16

NB = 80
EPT = NB * K
EPC = NS * EPT

NROWS = 10112
ZR = NROWS // NS
OR = 632
NPAD = NS * OR

_mesh = plsc.VectorSubcoreMesh(core_axis_name="c", subcore_axis_name="s")


def _sc_deg_body(dsta_hbm, z_hbm, ones_hbm, dega_hbm, degb_hbm,
                 idx_v, ones_v, deg_sh, sem):
    cid = lax.axis_index("c")
    sid = lax.axis_index("s")
    wid = cid * NS + sid
    pltpu.sync_copy(dsta_hbm.at[wid], idx_v)
    pltpu.sync_copy(ones_hbm, ones_v)
    pltpu.sync_copy(z_hbm, deg_sh.at[pl.ds(sid * ZR, ZR)])
    plsc.subcore_barrier()

    def chunk(g, carry):
        def fire(j, c):
            pltpu.async_copy(ones_v, deg_sh.at[idx_v.at[g * 8 + j]], sem,
                             add=True)
            return c

        lax.fori_loop(0, 8, fire, 0)

        def drain(j, c):
            pltpu.make_async_copy(ones_v, deg_sh.at[idx_v.at[g * 8 + j]],
                                  sem).wait()
            return c

        lax.fori_loop(0, 8, drain, 0)
        return carry

    lax.fori_loop(0, NB_A // 8, chunk, 0)
    plsc.subcore_barrier()

    @pl.when(cid == 0)
    def _():
        pltpu.sync_copy(deg_sh.at[pl.ds(sid * OR, OR)],
                        dega_hbm.at[pl.ds(sid * OR, OR)])

    @pl.when(cid == 1)
    def _():
        pltpu.sync_copy(deg_sh.at[pl.ds(sid * OR, OR)],
                        degb_hbm.at[pl.ds(sid * OR, OR)])


_sc_deg = functools.partial(
    pl.kernel,
    out_type=[jax.ShapeDtypeStruct((NPAD, CH), jnp.float32)] * 2,
    mesh=_mesh,
    scratch_types=[
        pltpu.VMEM((NB_A, K), jnp.int32),
        pltpu.VMEM((K, CH), jnp.float32),
        pltpu.VMEM_SHARED((NROWS, CH), jnp.float32),
        pltpu.SemaphoreType.DMA,
    ],
)(_sc_deg_body)


def _sc_agg_body(tbl0_hbm, tbl1_hbm, srct_hbm, dstt_hbm, z_hbm,
                 agg0_hbm, agg1_hbm, src_v, dst_v, rows_v, agg_sh, sem):
    cid = lax.axis_index("c")
    sid = lax.axis_index("s")
    pltpu.sync_copy(srct_hbm.at[sid], src_v)
    pltpu.sync_copy(dstt_hbm.at[sid], dst_v)
    pltpu.sync_copy(z_hbm, agg_sh.at[pl.ds(sid * ZR, ZR)])
    plsc.subcore_barrier()

    def step(b, carry):
        @pl.when(cid == 0)
        def _():
            pltpu.async_copy(tbl0_hbm.at[src_v.at[b]], rows_v, sem).wait()

        @pl.when(cid == 1)
        def _():
            pltpu.async_copy(tbl1_hbm.at[src_v.at[b]], rows_v, sem).wait()

        pltpu.sync_copy(rows_v, agg_sh.at[dst_v.at[b]], add=True)
        return carry

    lax.fori_loop(0, NB, step, 0)
    plsc.subcore_barrier()

    @pl.when(cid == 0)
    def _():
        pltpu.sync_copy(agg_sh.at[pl.ds(sid * OR, OR)],
                        agg0_hbm.at[pl.ds(sid * OR, OR)])

    @pl.when(cid == 1)
    def _():
        pltpu.sync_copy(agg_sh.at[pl.ds(sid * OR, OR)],
                        agg1_hbm.at[pl.ds(sid * OR, OR)])


_sc_agg = functools.partial(
    pl.kernel,
    out_type=[jax.ShapeDtypeStruct((NPAD, CH), jnp.float32)] * 2,
    mesh=_mesh,
    scratch_types=[
        pltpu.VMEM((NB, K), jnp.int32),
        pltpu.VMEM((NB, K), jnp.int32),
        pltpu.VMEM((K, CH), jnp.float32),
        pltpu.VMEM_SHARED((NROWS, CH), jnp.float32),
        pltpu.SemaphoreType.DMA,
    ],
)(_sc_agg_body)


R = 1000
GRID = N // R
_f32 = jnp.float32
_bf16 = jnp.bfloat16


def _bdot(a, b):
    return jnp.dot(a, b, preferred_element_type=_f32)


def _tc_prep_body(x_ref, da_ref, db_ref, w_ref, b_ref, o0, o1, od, oxr):
    deg = da_ref[...][:, :G16] + db_ref[...][:, :G16]
    dinv = lax.rsqrt(jnp.maximum(deg, 1.0))
    dv = dinv[:, 0:1]
    xx = x_ref[...]
    o0[...] = xx[:, :CH] * dv
    o1[...] = xx[:, CH:] * dv
    od[...] = dinv
    oxr[...] = _bdot(xx, w_ref[...]) + b_ref[...]


def _tc_prep(x, dega, degb, RW0, Rb0):
    return pl.pallas_call(
        _tc_prep_body,
        grid=(GRID,),
        in_specs=[
            pl.BlockSpec((R, C), lambda i: (i, 0)),
            pl.BlockSpec((R, CH), lambda i: (i, 0)),
            pl.BlockSpec((R, CH), lambda i: (i, 0)),
            pl.BlockSpec((C, C), lambda i: (0, 0)),
            pl.BlockSpec((1, C), lambda i: (0, 0)),
        ],
        out_specs=[
            pl.BlockSpec((R, CH), lambda i: (i, 0)),
            pl.BlockSpec((R, CH), lambda i: (i, 0)),
            pl.BlockSpec((R, G16), lambda i: (i, 0)),
            pl.BlockSpec((R, C), lambda i: (i, 0)),
        ],
        out_shape=[
            jax.ShapeDtypeStruct((N, CH), _f32),
            jax.ShapeDtypeStruct((N, CH), _f32),
            jax.ShapeDtypeStruct((N, G16), _f32),
            jax.ShapeDtypeStruct((N, C), _f32),
        ],
    )(x, dega, degb, RW0, Rb0.reshape(1, C))


def _tc_layer_body(a0_ref, a1_ref, dv_ref, xr_ref, w_ref, b_ref,
                   rw_ref, rb_ref, oh0, oh1, oxr):
    dv = dv_ref[...][:, 0:1]
    a0 = a0_ref[...] * dv
    a1 = a1_ref[...] * dv
    w = w_ref[...]
    t = _bdot(a0, w[:CH, :]) + _bdot(a1, w[CH:, :]) + b_ref[...]
    h = jax.nn.relu(jax.nn.relu(t) + xr_ref[...])
    oh0[...] = h[:, :CH] * dv
    oh1[...] = h[:, CH:] * dv
    oxr[...] = _bdot(h, rw_ref[...]) + rb_ref[...]


def _tc_layer(a0, a1, dinv16, xr0, W0, b0, RW1, Rb1):
    return pl.pallas_call(
        _tc_layer_body,
        grid=(GRID,),
        in_specs=[
            pl.BlockSpec((R, CH), lambda i: (i, 0)),
            pl.BlockSpec((R, CH), lambda i: (i, 0)),
            pl.BlockSpec((R, G16), lambda i: (i, 0)),
            pl.BlockSpec((R, C), lambda i: (i, 0)),
            pl.BlockSpec((C, C), lambda i: (0, 0)),
            pl.BlockSpec((1, C), lambda i: (0, 0)),
            pl.BlockSpec((C, C), lambda i: (0, 0)),
            pl.BlockSpec((1, C), lambda i: (0, 0)),
        ],
        out_specs=[
            pl.BlockSpec((R, CH), lambda i: (i, 0)),
            pl.BlockSpec((R, CH), lambda i: (i, 0)),
            pl.BlockSpec((R, C), lambda i: (i, 0)),
        ],
        out_shape=[
            jax.ShapeDtypeStruct((N, CH), _f32),
            jax.ShapeDtypeStruct((N, CH), _f32),
            jax.ShapeDtypeStruct((N, C), _f32),
        ],
    )(a0, a1, dinv16, xr0, W0, b0.reshape(1, C), RW1, Rb1.reshape(1, C))


def _tc_out_body(a0_ref, a1_ref, dv_ref, xr_ref, w_ref, b_ref, out_ref):
    dv = dv_ref[...][:, 0:1]
    a0 = a0_ref[...] * dv
    a1 = a1_ref[...] * dv
    w = w_ref[...]
    t = _bdot(a0, w[:CH, :]) + _bdot(a1, w[CH:, :]) + b_ref[...]
    out_ref[...] = jax.nn.relu(t) + xr_ref[...]


def _tc_out(a0, a1, dinv16, xr1, W1, b1):
    return pl.pallas_call(
        _tc_out_body,
        grid=(GRID,),
        in_specs=[
            pl.BlockSpec((R, CH), lambda i: (i, 0)),
            pl.BlockSpec((R, CH), lambda i: (i, 0)),
            pl.BlockSpec((R, G16), lambda i: (i, 0)),
            pl.BlockSpec((R, C), lambda i: (i, 0)),
            pl.BlockSpec((C, C), lambda i: (0, 0)),
            pl.BlockSpec((1, C), lambda i: (0, 0)),
        ],
        out_specs=pl.BlockSpec((R, C), lambda i: (i, 0)),
        out_shape=jax.ShapeDtypeStruct((N, C), _f32),
    )(a0, a1, dinv16, xr1, W1, b1.reshape(1, C))


def kernel(x, edge_index_K, W0, b0, W1, b1, RW0, Rb0, RW1, Rb1):
    src = edge_index_K[0]
    dst = edge_index_K[1]

    dsta = jnp.concatenate(
        [dst, jnp.full((EPA - E,), N, jnp.int32)]).reshape(NW, NB_A, K)
    srct = jnp.concatenate(
        [src, jnp.zeros((EPC - E,), jnp.int32)]).reshape(NS, NB, K)
    dstt = jnp.concatenate(
        [dst, jnp.full((EPC - E,), N, jnp.int32)]).reshape(NS, NB, K)

    ones128 = jnp.ones((K, CH), _f32)
    z128 = jnp.zeros((ZR, CH), _f32)

    dega, degb = _sc_deg(dsta, z128, ones128)
    xs0h0, xs0h1, dinv16, xr0 = _tc_prep(x, dega, degb, RW0, Rb0)
    agg0h0, agg0h1 = _sc_agg(xs0h0, xs0h1, srct, dstt, z128)
    hsh0, hsh1, xr1 = _tc_layer(agg0h0, agg0h1, dinv16, xr0, W0, b0, RW1, Rb1)
    agg1h0, agg1h1 = _sc_agg(hsh0, hsh1, srct, dstt, z128)
    return _tc_out(agg1h0, agg1h1, dinv16, xr1, W1, b1)


# speedup vs baseline: 1.2466x; 1.2466x over previous
import functools

import jax
import jax.numpy as jnp
from jax import lax
from jax.experimental import pallas as pl
from jax.experimental.pallas import tpu as pltpu, tpu_sc as plsc

N = 10000
C = 256
E = 160000
CH = 128
NC, NS, K = 2, 16, 128
NW = NC * NS

NB_A = 40
EPA = NW * NB_A * K
G16 = 16

NB = 79
EPT = NB * K
EPC = NS * EPT

NROWS = 10240
ZR = NROWS // NS
OR = 632
NPAD = NS * OR

_mesh = plsc.VectorSubcoreMesh(core_axis_name="c", subcore_axis_name="s")


def _sc_deg_body(dsta_hbm, z_hbm, ones_hbm, dega_hbm, degb_hbm,
                 idx_v, ones_v, deg_sh, sem):
    cid = lax.axis_index("c")
    sid = lax.axis_index("s")
    wid = cid * NS + sid
    pltpu.sync_copy(dsta_hbm.at[wid], idx_v)
    pltpu.sync_copy(ones_hbm, ones_v)
    pltpu.sync_copy(z_hbm, deg_sh.at[pl.ds(sid * ZR, ZR)])
    plsc.subcore_barrier()

    def chunk(g, carry):
        def fire(j, c):
            pltpu.async_copy(ones_v, deg_sh.at[idx_v.at[g * 8 + j]], sem,
                             add=True)
            return c

        lax.fori_loop(0, 8, fire, 0)

        def drain(j, c):
            pltpu.make_async_copy(ones_v, deg_sh.at[idx_v.at[g * 8 + j]],
                                  sem).wait()
            return c

        lax.fori_loop(0, 8, drain, 0)
        return carry

    lax.fori_loop(0, NB_A // 8, chunk, 0)
    plsc.subcore_barrier()

    @pl.when(cid == 0)
    def _():
        pltpu.sync_copy(deg_sh.at[pl.ds(sid * OR, OR)],
                        dega_hbm.at[pl.ds(sid * OR, OR)])

    @pl.when(cid == 1)
    def _():
        pltpu.sync_copy(deg_sh.at[pl.ds(sid * OR, OR)],
                        degb_hbm.at[pl.ds(sid * OR, OR)])


_sc_deg = functools.partial(
    pl.kernel,
    out_type=[jax.ShapeDtypeStruct((NPAD, CH), jnp.float32)] * 2,
    mesh=_mesh,
    scratch_types=[
        pltpu.VMEM((NB_A, K), jnp.int32),
        pltpu.VMEM((K, CH), jnp.float32),
        pltpu.VMEM_SHARED((NROWS, CH), jnp.float32),
        pltpu.SemaphoreType.DMA,
    ],
)(_sc_deg_body)


def _sc_agg_body(tbl0_hbm, tbl1_hbm, srct_hbm, dstt_hbm, z_hbm,
                 agg0_hbm, agg1_hbm, src_v, dst_v, rows_v, agg_sh, sem):
    cid = lax.axis_index("c")
    sid = lax.axis_index("s")
    pltpu.sync_copy(srct_hbm.at[sid], src_v)
    pltpu.sync_copy(dstt_hbm.at[sid], dst_v)
    pltpu.sync_copy(z_hbm, agg_sh.at[pl.ds(sid * ZR, ZR)])
    plsc.subcore_barrier()

    def step(b, carry):
        @pl.when(cid == 0)
        def _():
            pltpu.async_copy(tbl0_hbm.at[src_v.at[b]], rows_v, sem).wait()

        @pl.when(cid == 1)
        def _():
            pltpu.async_copy(tbl1_hbm.at[src_v.at[b]], rows_v, sem).wait()

        pltpu.sync_copy(rows_v, agg_sh.at[dst_v.at[b]], add=True)
        return carry

    lax.fori_loop(0, NB, step, 0)
    plsc.subcore_barrier()

    @pl.when(cid == 0)
    def _():
        pltpu.sync_copy(agg_sh.at[pl.ds(sid * OR, OR)],
                        agg0_hbm.at[pl.ds(sid * OR, OR)])

    @pl.when(cid == 1)
    def _():
        pltpu.sync_copy(agg_sh.at[pl.ds(sid * OR, OR)],
                        agg1_hbm.at[pl.ds(sid * OR, OR)])


_sc_agg = functools.partial(
    pl.kernel,
    out_type=[jax.ShapeDtypeStruct((NPAD, CH), jnp.float32)] * 2,
    mesh=_mesh,
    scratch_types=[
        pltpu.VMEM((NB, K), jnp.int32),
        pltpu.VMEM((NB, K), jnp.int32),
        pltpu.VMEM((K, CH), jnp.float32),
        pltpu.VMEM_SHARED((NROWS, CH), jnp.float32),
        pltpu.SemaphoreType.DMA,
    ],
)(_sc_agg_body)


R = 1000
GRID = N // R
_f32 = jnp.float32
_bf16 = jnp.bfloat16


def _bdot(a, b):
    return jnp.dot(a, b, preferred_element_type=_f32)


def _tc_prep_body(x_ref, da_ref, db_ref, w_ref, b_ref, o0, o1, od, oxr):
    deg = da_ref[...][:, :G16] + db_ref[...][:, :G16]
    dinv = lax.rsqrt(jnp.maximum(deg, 1.0))
    dv = dinv[:, 0:1]
    xx = x_ref[...]
    o0[...] = xx[:, :CH] * dv
    o1[...] = xx[:, CH:] * dv
    od[...] = dinv
    oxr[...] = _bdot(xx, w_ref[...]) + b_ref[...]


def _tc_prep(x, dega, degb, RW0, Rb0):
    return pl.pallas_call(
        _tc_prep_body,
        grid=(GRID,),
        in_specs=[
            pl.BlockSpec((R, C), lambda i: (i, 0)),
            pl.BlockSpec((R, CH), lambda i: (i, 0)),
            pl.BlockSpec((R, CH), lambda i: (i, 0)),
            pl.BlockSpec((C, C), lambda i: (0, 0)),
            pl.BlockSpec((1, C), lambda i: (0, 0)),
        ],
        out_specs=[
            pl.BlockSpec((R, CH), lambda i: (i, 0)),
            pl.BlockSpec((R, CH), lambda i: (i, 0)),
            pl.BlockSpec((R, G16), lambda i: (i, 0)),
            pl.BlockSpec((R, C), lambda i: (i, 0)),
        ],
        out_shape=[
            jax.ShapeDtypeStruct((N, CH), _f32),
            jax.ShapeDtypeStruct((N, CH), _f32),
            jax.ShapeDtypeStruct((N, G16), _f32),
            jax.ShapeDtypeStruct((N, C), _f32),
        ],
    )(x, dega, degb, RW0, Rb0.reshape(1, C))


def _tc_layer_body(a0_ref, a1_ref, dv_ref, xr_ref, w_ref, b_ref,
                   rw_ref, rb_ref, oh0, oh1, oxr):
    dv = dv_ref[...][:, 0:1]
    a0 = a0_ref[...] * dv
    a1 = a1_ref[...] * dv
    w = w_ref[...]
    t = _bdot(a0, w[:CH, :]) + _bdot(a1, w[CH:, :]) + b_ref[...]
    h = jax.nn.relu(jax.nn.relu(t) + xr_ref[...])
    oh0[...] = h[:, :CH] * dv
    oh1[...] = h[:, CH:] * dv
    oxr[...] = _bdot(h, rw_ref[...]) + rb_ref[...]


def _tc_layer(a0, a1, dinv16, xr0, W0, b0, RW1, Rb1):
    return pl.pallas_call(
        _tc_layer_body,
        grid=(GRID,),
        in_specs=[
            pl.BlockSpec((R, CH), lambda i: (i, 0)),
            pl.BlockSpec((R, CH), lambda i: (i, 0)),
            pl.BlockSpec((R, G16), lambda i: (i, 0)),
            pl.BlockSpec((R, C), lambda i: (i, 0)),
            pl.BlockSpec((C, C), lambda i: (0, 0)),
            pl.BlockSpec((1, C), lambda i: (0, 0)),
            pl.BlockSpec((C, C), lambda i: (0, 0)),
            pl.BlockSpec((1, C), lambda i: (0, 0)),
        ],
        out_specs=[
            pl.BlockSpec((R, CH), lambda i: (i, 0)),
            pl.BlockSpec((R, CH), lambda i: (i, 0)),
            pl.BlockSpec((R, C), lambda i: (i, 0)),
        ],
        out_shape=[
            jax.ShapeDtypeStruct((N, CH), _f32),
            jax.ShapeDtypeStruct((N, CH), _f32),
            jax.ShapeDtypeStruct((N, C), _f32),
        ],
    )(a0, a1, dinv16, xr0, W0, b0.reshape(1, C), RW1, Rb1.reshape(1, C))


def _tc_out_body(a0_ref, a1_ref, dv_ref, xr_ref, w_ref, b_ref, out_ref):
    dv = dv_ref[...][:, 0:1]
    a0 = a0_ref[...] * dv
    a1 = a1_ref[...] * dv
    w = w_ref[...]
    t = _bdot(a0, w[:CH, :]) + _bdot(a1, w[CH:, :]) + b_ref[...]
    out_ref[...] = jax.nn.relu(t) + xr_ref[...]


def _tc_out(a0, a1, dinv16, xr1, W1, b1):
    return pl.pallas_call(
        _tc_out_body,
        grid=(GRID,),
        in_specs=[
            pl.BlockSpec((R, CH), lambda i: (i, 0)),
            pl.BlockSpec((R, CH), lambda i: (i, 0)),
            pl.BlockSpec((R, G16), lambda i: (i, 0)),
            pl.BlockSpec((R, C), lambda i: (i, 0)),
            pl.BlockSpec((C, C), lambda i: (0, 0)),
            pl.BlockSpec((1, C), lambda i: (0, 0)),
        ],
        out_specs=pl.BlockSpec((R, C), lambda i: (i, 0)),
        out_shape=jax.ShapeDtypeStruct((N, C), _f32),
    )(a0, a1, dinv16, xr1, W1, b1.reshape(1, C))


def kernel(x, edge_index_K, W0, b0, W1, b1, RW0, Rb0, RW1, Rb1):
    src = edge_index_K[0]
    dst = edge_index_K[1]

    dsta = jnp.concatenate(
        [dst, jnp.full((EPA - E,), N, jnp.int32)]).reshape(NW, NB_A, K)
    srct = jnp.concatenate(
        [src, jnp.zeros((EPC - E,), jnp.int32)]).reshape(NS, NB, K)
    dstt = jnp.concatenate(
        [dst, jnp.full((EPC - E,), N, jnp.int32)]).reshape(NS, NB, K)

    ones128 = jnp.ones((K, CH), _f32)
    z128 = jnp.zeros((ZR, CH), _f32)

    dega, degb = _sc_deg(dsta, z128, ones128)
    xs0h0, xs0h1, dinv16, xr0 = _tc_prep(x, dega, degb, RW0, Rb0)
    agg0h0, agg0h1 = _sc_agg(xs0h0, xs0h1, srct, dstt, z128)
    hsh0, hsh1, xr1 = _tc_layer(agg0h0, agg0h1, dinv16, xr0, W0, b0, RW1, Rb1)
    agg1h0, agg1h1 = _sc_agg(hsh0, hsh1, srct, dstt, z128)
    return _tc_out(agg1h0, agg1h1, dinv16, xr1, W1, b1)
